# Initial kernel scaffold; baseline (speedup 1.0000x reference)
import jax
import jax.numpy as jnp
from jax.experimental import pallas as pl


def kernel(hidden_states, state, Wq, Wk):
    raise NotImplementedError("write your pallas kernel here")


# speedup vs baseline: 11.8147x; 11.8147x over previous
import functools

import jax
import jax.numpy as jnp
from jax.experimental import pallas as pl
from jax.experimental.pallas import tpu as pltpu

B = 8
L = 2048
D = 1024
S = 256
C = L // S


def _body(h_ref, wq_ref, wk_ref, o_ref, carry_ref, qprev_ref, qqprev_ref):
    c = pl.program_id(1)
    h = h_ref[0]
    wq = wq_ref[...]
    wk = wk_ref[...]
    dn = (((1,), (1,)), ((), ()))
    q = jax.lax.dot_general(h, wq, dn, precision=jax.lax.Precision.DEFAULT,
                            preferred_element_type=jnp.float32)
    k = jax.lax.dot_general(h, wk, dn, precision=jax.lax.Precision.DEFAULT,
                            preferred_element_type=jnp.float32)
    qq = jnp.sum(q * q, axis=1, keepdims=True)
    kk = jnp.sum(k * k, axis=1, keepdims=True)
    q_sh = jnp.concatenate([qprev_ref[...], q[:-1]], axis=0)
    qq_sh = jnp.concatenate([qqprev_ref[...], qq[:-1]], axis=0)
    cross = jnp.sum(q_sh * k, axis=1, keepdims=True)
    eps = 1e-12
    denom = jnp.maximum(jnp.sqrt(qq_sh), eps) * jnp.maximum(jnp.sqrt(kk), eps)
    cos = cross / denom
    prob = jnp.clip((1.0 - cos) * 0.5, 0.0, 1.0)
    row = jax.lax.broadcasted_iota(jnp.int32, (S, 1), 0)
    prob = jnp.where(jnp.logical_and(c == 0, row == 0), 1.0, prob)
    mask = prob > 0.5
    a = jnp.where(mask, 1.0 - prob, 1.0)
    bv = jnp.where(mask, prob, 0.0) * h
    stride = 1
    while stride < S:
        a_sh = jnp.concatenate(
            [jnp.ones((stride, 1), jnp.float32), a[:-stride]], axis=0)
        b_sh = jnp.concatenate(
            [jnp.zeros((stride, D), jnp.float32), bv[:-stride]], axis=0)
        bv = bv + a * b_sh
        a = a * a_sh
        stride *= 2

    @pl.when(c == 0)
    def _zero_carry():
        carry_ref[...] = jnp.zeros_like(carry_ref)

    carry = carry_ref[...]
    full = bv + a * carry
    o_ref[0] = h + full
    carry_ref[...] = full[-1:]
    qprev_ref[...] = q[-1:]
    qqprev_ref[...] = qq[-1:]


@jax.jit
def kernel(hidden_states, state, Wq, Wk):
    del state
    grid = (B, C)
    out = pl.pallas_call(
        _body,
        grid=grid,
        in_specs=[
            pl.BlockSpec((1, S, D), lambda b, c: (b, c, 0)),
            pl.BlockSpec((D, D), lambda b, c: (0, 0)),
            pl.BlockSpec((D, D), lambda b, c: (0, 0)),
        ],
        out_specs=pl.BlockSpec((1, S, D), lambda b, c: (b, c, 0)),
        out_shape=jax.ShapeDtypeStruct((B, L, D), jnp.float32),
        scratch_shapes=[
            pltpu.VMEM((1, D), jnp.float32),
            pltpu.VMEM((1, D), jnp.float32),
            pltpu.VMEM((1, 1), jnp.float32),
        ],
    )(hidden_states, Wq, Wk)
    return out


# speedup vs baseline: 11.8819x; 1.0057x over previous
import functools

import jax
import jax.numpy as jnp
from jax.experimental import pallas as pl
from jax.experimental.pallas import tpu as pltpu

B = 8
L = 2048
D = 1024
S = 256
C = L // S


def _body(h_ref, wq_ref, wk_ref, o_ref, carry_ref, qprev_ref, qqprev_ref):
    c = pl.program_id(1)
    h = h_ref[0]
    wq = wq_ref[...]
    wk = wk_ref[...]
    dn = (((1,), (1,)), ((), ()))
    q = jax.lax.dot_general(h, wq, dn, precision=jax.lax.Precision.DEFAULT,
                            preferred_element_type=jnp.float32)
    k = jax.lax.dot_general(h, wk, dn, precision=jax.lax.Precision.DEFAULT,
                            preferred_element_type=jnp.float32)
    qq = jnp.sum(q * q, axis=1, keepdims=True)
    kk = jnp.sum(k * k, axis=1, keepdims=True)
    q_sh = jnp.concatenate([qprev_ref[...], q[:-1]], axis=0)
    qq_sh = jnp.concatenate([qqprev_ref[...], qq[:-1]], axis=0)
    cross = jnp.sum(q_sh * k, axis=1, keepdims=True)
    eps = 1e-12
    denom = jnp.maximum(jnp.sqrt(qq_sh), eps) * jnp.maximum(jnp.sqrt(kk), eps)
    cos = cross / denom
    prob = jnp.clip((1.0 - cos) * 0.5, 0.0, 1.0)
    row = jax.lax.broadcasted_iota(jnp.int32, (S, 1), 0)
    prob = jnp.where(jnp.logical_and(c == 0, row == 0), 1.0, prob)
    mask = prob > 0.5
    a = jnp.where(mask, 1.0 - prob, 1.0)
    bv = jnp.where(mask, prob, 0.0) * h
    stride = 1
    while stride < S:
        a_sh = jnp.concatenate(
            [jnp.ones((stride, 1), jnp.float32), a[:-stride]], axis=0)
        b_sh = jnp.concatenate(
            [jnp.zeros((stride, D), jnp.float32), bv[:-stride]], axis=0)
        bv = bv + a * b_sh
        a = a * a_sh
        stride *= 2

    @pl.when(c == 0)
    def _zero_carry():
        carry_ref[...] = jnp.zeros_like(carry_ref)

    carry = carry_ref[...]
    full = bv + a * carry
    o_ref[0] = h + full
    carry_ref[...] = full[-1:]
    qprev_ref[...] = q[-1:]
    qqprev_ref[...] = qq[-1:]


@jax.jit
def kernel(hidden_states, state, Wq, Wk):
    del state
    grid = (B, C)
    out = pl.pallas_call(
        _body,
        grid=grid,
        in_specs=[
            pl.BlockSpec((1, S, D), lambda b, c: (b, c, 0)),
            pl.BlockSpec((D, D), lambda b, c: (0, 0)),
            pl.BlockSpec((D, D), lambda b, c: (0, 0)),
        ],
        out_specs=pl.BlockSpec((1, S, D), lambda b, c: (b, c, 0)),
        out_shape=jax.ShapeDtypeStruct((B, L, D), jnp.float32),
        scratch_shapes=[
            pltpu.VMEM((1, D), jnp.float32),
            pltpu.VMEM((1, D), jnp.float32),
            pltpu.VMEM((1, 1), jnp.float32),
        ],
        compiler_params=pltpu.CompilerParams(
            dimension_semantics=("parallel", "arbitrary")),
    )(hidden_states, Wq, Wk)
    return out
